# baseline (speedup 1.0000x reference)
import functools

import numpy as np
import jax
import jax.numpy as jnp
from jax.experimental import pallas as pl
from jax.experimental.pallas import tpu as pltpu

_N = 4096
_M = 512
_IN = 128
_HID = 192
_OUT = 128
_K = 6
_NEG = -jnp.inf


def _build_er():
    rng = np.random.default_rng(12345)
    u = rng.random((_M, _M))
    iu = np.triu(np.ones((_M, _M), dtype=bool), 1)
    mask = (u < 0.1) & iu
    r, c = np.nonzero(mask)
    src = np.concatenate([r, c]).astype(np.int32)
    dst = np.concatenate([c, r]).astype(np.int32)
    deg = np.bincount(dst, minlength=_M).astype(np.int32)
    dmax = int(deg.max())
    nbr = np.zeros((_M, dmax), np.int32)
    fill = np.zeros(_M, np.int32)
    for s, d in zip(src, dst):
        nbr[d, fill[d]] = s
        fill[d] += 1
    nbr_b = np.broadcast_to(nbr.T[:, :, None], (dmax, _M, 128)).copy()
    deg_b = np.broadcast_to(deg[:, None], (_M, 128)).copy()
    return nbr_b, deg_b, dmax


_NBR_B, _DEG_B, _DMAX = _build_er()


def _hm_perm(out_dim, nh):
    hs = out_dim // nh
    p = np.empty(out_dim, np.int64)
    for g in range(nh):
        for s in range(hs):
            p[g * hs + s] = s * nh + g
    return p


_P1 = _hm_perm(2 * _HID, 3)
_P2 = _hm_perm(_HID, 2)
_P3 = _hm_perm(_OUT, 2)
_PM = np.zeros((_OUT, _OUT), np.float32)
_PM[np.arange(_OUT), _P3] = 1.0


def _bb(v, w):
    return jnp.broadcast_to(v.reshape(1, w), (8, w))


def _dmon_body(x_ref, w_ref, bc_ref, s_ref, pooled_ref, csum_ref):
    i = pl.program_id(0)
    x = x_ref[...]
    lt = jax.lax.dot_general(w_ref[...], x, (((0,), (1,)), ((), ())),
                             preferred_element_type=jnp.float32)
    lt = lt + bc_ref[:, 0:1]
    m = jnp.max(lt, axis=0, keepdims=True)
    e = jnp.exp(lt - m)
    st = e / jnp.sum(e, axis=0, keepdims=True)
    s_ref[...] = st

    @pl.when(i == 0)
    def _():
        pooled_ref[...] = jnp.zeros_like(pooled_ref)
        csum_ref[...] = jnp.zeros_like(csum_ref)

    pooled_ref[...] += jax.lax.dot_general(
        st, x, (((1,), (0,)), ((), ())), preferred_element_type=jnp.float32)
    csum_ref[...] += jnp.broadcast_to(
        jnp.sum(st, axis=1, keepdims=True), csum_ref.shape)


def _run_dmon(x, w, bc):
    nb = _N // _M
    return pl.pallas_call(
        _dmon_body,
        grid=(nb,),
        in_specs=[
            pl.BlockSpec((_M, _IN), lambda i: (i, 0)),
            pl.BlockSpec((_IN, _M), lambda i: (0, 0)),
            pl.BlockSpec((_M, 128), lambda i: (0, 0)),
        ],
        out_specs=[
            pl.BlockSpec((_M, _M), lambda i: (0, i)),
            pl.BlockSpec((_M, _IN), lambda i: (0, 0)),
            pl.BlockSpec((_M, 128), lambda i: (0, 0)),
        ],
        out_shape=[
            jax.ShapeDtypeStruct((_M, _N), jnp.float32),
            jax.ShapeDtypeStruct((_M, _IN), jnp.float32),
            jax.ShapeDtypeStruct((_M, 128), jnp.float32),
        ],
    )(x, w, bc)


def _onehot(idx1):
    lane = jax.lax.broadcasted_iota(jnp.int32, (_M, _M), 1)
    eq = lane == jnp.broadcast_to(idx1, (_M, _M))
    return jnp.where(eq, jnp.float32(1.0), jnp.float32(0.0))


def _gath(tab, idx1):
    return jnp.dot(_onehot(idx1), tab, preferred_element_type=jnp.float32)


def _conv1_body(pooled_ref, csum_ref, wsrc_ref, wqd_ref, bq_ref,
                wkd_ref, bk_ref, w1d_ref, b1_ref,
                w2p_ref, nbr_ref, deg_ref, y1_ref,
                bq_s, bk_s, ud, x0s):
    k = pl.program_id(0)
    d_out = 2 * _HID

    @pl.when(k == 0)
    def _():
        pv = pooled_ref[...]
        alpha = jnp.float32(1.6732632423543772)
        scl = jnp.float32(1.0507009873554805)
        selu = scl * jnp.where(pv > 0, pv, alpha * (jnp.exp(pv) - 1.0))
        x0 = selu / csum_ref[...]
        x0s[...] = x0
        bq_s[...] = jnp.dot(x0, wqd_ref[...],
                            preferred_element_type=jnp.float32) + bq_ref[0:1, :]
        bk_s[...] = jnp.dot(x0, wkd_ref[...],
                            preferred_element_type=jnp.float32) + bk_ref[0:1, :]
        ud[...] = jnp.dot(x0, w1d_ref[...],
                          preferred_element_type=jnp.float32) + b1_ref[0:1, :]

    idx1 = nbr_ref[0, :, 0:1]
    xs = _gath(x0s[...], idx1)
    g_all = jnp.dot(xs, wsrc_ref[...], preferred_element_type=jnp.float32)
    q = g_all[:, :d_out] + bq_s[...]
    kk = g_all[:, d_out:2 * d_out] + bk_s[...]
    scale = jnp.sqrt(jnp.float32(d_out))
    hs = 128
    ew = [[jnp.sum(q[:, h * hs:(h + 1) * hs] * kk[:, g * hs:(g + 1) * hs],
                   axis=1, keepdims=True) / scale
           for g in range(3)] for h in range(3)]
    pre = ud[...] + g_all[:, 2 * d_out:]
    o1 = jax.nn.relu(pre) + x0s[...]
    m2 = jnp.dot(o1, w2p_ref[...], preferred_element_type=jnp.float32)
    fin = jnp.concatenate(
        [sum(m2[:, h * hs:(h + 1) * hs] * ew[h][g] for h in range(3))
         for g in range(3)], axis=1)
    mask = jnp.broadcast_to(k < deg_ref[:, 0:1], (_M, d_out))
    cur = jnp.where(mask, fin, _NEG)

    @pl.when(k == 0)
    def _():
        y1_ref[...] = cur

    @pl.when(k > 0)
    def _():
        y1_ref[...] = jnp.maximum(y1_ref[...], cur)

    @pl.when(k == _DMAX - 1)
    def _():
        v = y1_ref[...]
        y1_ref[...] = jnp.where(jnp.isfinite(v), v, jnp.float32(0.0))


def _run_conv1(pooled, csum, wsrc, wqd, bq, wkd, bk, w1d, b1, w2p,
               nbr_b, deg_b):
    d_out = 2 * _HID
    wsrc_w = 2 * d_out + _IN
    full = lambda r, c: pl.BlockSpec((r, c), lambda k: (0, 0))
    return pl.pallas_call(
        _conv1_body,
        grid=(_DMAX,),
        in_specs=[
            full(_M, _IN), full(_M, 128),
            full(_IN, wsrc_w), full(_IN, d_out), full(8, d_out),
            full(_IN, d_out), full(8, d_out),
            full(_IN, _IN), full(8, _IN),
            full(_IN, d_out),
            pl.BlockSpec((1, _M, 128), lambda k: (k, 0, 0)),
            full(_M, 128),
        ],
        out_specs=pl.BlockSpec((_M, d_out), lambda k: (0, 0)),
        out_shape=jax.ShapeDtypeStruct((_M, d_out), jnp.float32),
        scratch_shapes=[
            pltpu.VMEM((_M, d_out), jnp.float32),
            pltpu.VMEM((_M, d_out), jnp.float32),
            pltpu.VMEM((_M, _IN), jnp.float32),
            pltpu.VMEM((_M, _IN), jnp.float32),
        ],
    )(pooled, csum, wsrc, wqd, bq, wkd, bk, w1d, b1, w2p, nbr_b, deg_b)


def _knn_body(y_ref, idx_ref):
    y = y_ref[...]
    d = y.shape[1]
    sqc = jnp.sum(y * y, axis=1, keepdims=True)
    sqr = jax.lax.dot_general(jnp.ones((1, d), jnp.float32), y * y,
                              (((1,), (1,)), ((), ())),
                              preferred_element_type=jnp.float32)
    g = jax.lax.dot_general(y, y, (((1,), (1,)), ((), ())),
                            preferred_element_type=jnp.float32)
    dist = sqc + sqr - 2.0 * g
    ii = jax.lax.broadcasted_iota(jnp.int32, (_M, _M), 0)
    jj = jax.lax.broadcasted_iota(jnp.int32, (_M, _M), 1)
    dist = jnp.where(ii == jj, jnp.inf, dist)
    lane = jax.lax.broadcasted_iota(jnp.int32, (_M, 128), 1)
    acc = jnp.zeros((_M, 128), jnp.int32)
    for k in range(_K):
        m = jnp.min(dist, axis=1, keepdims=True)
        cand = jnp.where(dist == m, jj, jnp.int32(_N))
        j = jnp.min(cand, axis=1, keepdims=True)
        acc = jnp.where(lane == k, jnp.broadcast_to(j, (_M, 128)), acc)
        dist = jnp.where(jj == jnp.broadcast_to(j, (_M, _M)), jnp.inf, dist)
    idx_ref[...] = acc


def _run_knn(y):
    return pl.pallas_call(
        _knn_body,
        out_shape=jax.ShapeDtypeStruct((_M, 128), jnp.int32),
    )(y)


def _convk_body(y_ref, idx_ref, wsrc_ref, wqd_ref, bq_ref, wkd_ref,
                bk_ref, w1d_ref, b1_ref, w2p_ref, nw_ref, nb_ref,
                nms_ref, out_ref, *, din, dout, nh, gather_first):
    y = y_ref[...]
    if not gather_first:
        tsrc = jnp.dot(y, wsrc_ref[...], preferred_element_type=jnp.float32)
    bq = jnp.dot(y, wqd_ref[...],
                 preferred_element_type=jnp.float32) + bq_ref[0:1, :]
    bk = jnp.dot(y, wkd_ref[...],
                 preferred_element_type=jnp.float32) + bk_ref[0:1, :]
    ud = jnp.dot(y, w1d_ref[...],
                 preferred_element_type=jnp.float32) + b1_ref[0:1, :]
    hs = dout // nh
    scale = jnp.sqrt(jnp.float32(dout))
    acc = jnp.zeros((_M, dout), jnp.float32)
    for k in range(_K):
        idx1 = idx_ref[:, k:k + 1]
        if gather_first:
            ys = _gath(y, idx1)
            g_all = jnp.dot(ys, wsrc_ref[...],
                            preferred_element_type=jnp.float32)
        else:
            g_all = _gath(tsrc, idx1)
        q = g_all[:, :dout] + bq
        kk = g_all[:, dout:2 * dout] + bk
        ew = [[jnp.sum(q[:, h * hs:(h + 1) * hs] * kk[:, g * hs:(g + 1) * hs],
                       axis=1, keepdims=True) / scale
               for g in range(nh)] for h in range(nh)]
        pre = ud + g_all[:, 2 * dout:]
        o1 = jax.nn.relu(pre) + y
        m2 = jnp.dot(o1, w2p_ref[...], preferred_element_type=jnp.float32)
        fin = jnp.concatenate(
            [sum(m2[:, h * hs:(h + 1) * hs] * ew[h][g] for h in range(nh))
             for g in range(nh)], axis=1)
        acc = acc + fin
    ym = acc / jnp.float32(_K)
    mean = jnp.mean(ym, axis=0, keepdims=True)
    out = ym - nms_ref[0:1, :] * mean
    var = jnp.mean(out * out, axis=0, keepdims=True)
    out_ref[...] = nw_ref[0:1, :] * out / jnp.sqrt(var + 1e-5) + nb_ref[0:1, :]


def _run_convk(y, idx, wsrc, wqd, bq, wkd, bk, w1d, b1, w2p,
               nw, nb, nms, din, dout, nh, gather_first):
    body = functools.partial(_convk_body, din=din, dout=dout, nh=nh,
                             gather_first=gather_first)
    return pl.pallas_call(
        body,
        out_shape=jax.ShapeDtypeStruct((_M, dout), jnp.float32),
    )(y, idx, wsrc, wqd, bq, wkd, bk, w1d, b1, w2p, nw, nb, nms)


def _tail_body(y_ref, idx_ref, pm_ref, wg_ref, bg_ref, yout_ref, z_ref):
    y = y_ref[...]
    agg = jnp.zeros((_M, _OUT), jnp.float32)
    for k in range(_K):
        agg = agg + _gath(y, idx_ref[:, k:k + 1])
    h4 = y + agg
    yout_ref[...] = jnp.dot(y, pm_ref[...], preferred_element_type=jnp.float32)
    logits = jnp.dot(h4, wg_ref[...],
                     preferred_element_type=jnp.float32) + bg_ref[0:1, :]
    m = jnp.max(logits, axis=1, keepdims=True)
    e = jnp.exp(logits - m)
    z_ref[...] = e / jnp.sum(e, axis=1, keepdims=True)


def _run_tail(y, idx, pm, wg, bg):
    return pl.pallas_call(
        _tail_body,
        out_shape=[
            jax.ShapeDtypeStruct((_M, _OUT), jnp.float32),
            jax.ShapeDtypeStruct((_M, 128), jnp.float32),
        ],
    )(y, idx, pm, wg, bg)


def kernel(x, edge_index, batch, params):
    p = params
    del edge_index, batch
    f32 = jnp.float32
    p1, p2, p3 = _P1, _P2, _P3

    w_dmon = p['dmon_w1'] @ p['dmon_w2']
    b_dmon = p['dmon_b1'] @ p['dmon_w2'] + p['dmon_b2']
    bc = jnp.broadcast_to(b_dmon[:, None], (_M, 128))

    s_out, pooled, csum = _run_dmon(x, w_dmon, bc)

    d1 = 2 * _HID
    wq = p['c1_wq1'] @ p['c1_wq2']
    bq = p['c1_bq1'] @ p['c1_wq2'] + p['c1_bq2']
    wk = p['c1_wk1'] @ p['c1_wk2']
    bk = p['c1_bk1'] @ p['c1_wk2'] + p['c1_bk2']
    w2p1 = p['c1_w2'][:, p1]
    wsrc1 = jnp.concatenate([wq[:_IN], wk[:_IN], p['c1_w1'][_IN:]], axis=1)
    y1 = _run_conv1(
        pooled, csum,
        wsrc1, wq[_IN:], _bb(bq, d1),
        wk[_IN:], _bb(bk, d1),
        p['c1_w1'][:_IN], _bb(p['c1_b1'], _IN),
        w2p1, jnp.asarray(_NBR_B), jnp.asarray(_DEG_B))

    idx2 = _run_knn(y1)
    wsrc2 = jnp.concatenate(
        [p['c2_wq'][:d1][p1], p['c2_wk'][:d1][p1],
         p['c2_w1'][d1:][p1][:, p1]], axis=1)
    y2 = _run_convk(
        y1, idx2,
        wsrc2, p['c2_wq'][d1:][p1], _bb(p['c2_bq'], _HID),
        p['c2_wk'][d1:][p1], _bb(p['c2_bk'], _HID),
        p['c2_w1'][:d1][p1][:, p1],
        _bb(p['c2_b1'][p1], d1),
        p['c2_w2'][p1][:, p2],
        _bb(p['n2_w'][p2], _HID), _bb(p['n2_b'][p2], _HID),
        _bb(p['n2_ms'][p2], _HID),
        din=d1, dout=_HID, nh=2, gather_first=False)

    idx3 = _run_knn(y2)
    wsrc3 = jnp.concatenate(
        [p['c3_wq'][:_HID][p2], p['c3_wk'][:_HID][p2],
         p['c3_w1'][_HID:][p2][:, p2]], axis=1)
    y3 = _run_convk(
        y2, idx3,
        wsrc3, p['c3_wq'][_HID:][p2], _bb(p['c3_bq'], _OUT),
        p['c3_wk'][_HID:][p2], _bb(p['c3_bk'], _OUT),
        p['c3_w1'][:_HID][p2][:, p2],
        _bb(p['c3_b1'][p2], _HID),
        p['c3_w2'][p2][:, p3],
        _bb(p['n3_w'][p3], _OUT), _bb(p['n3_b'][p3], _OUT),
        _bb(p['n3_ms'][p3], _OUT),
        din=_HID, dout=_OUT, nh=2, gather_first=True)

    wg = p['g_w1'] @ p['g_w2']
    bg = p['g_b1'] @ p['g_w2'] + p['g_b2']
    pm = jnp.asarray(_PM)
    wg_p = jnp.pad(pm @ wg, ((0, 0), (0, 128 - wg.shape[1])))
    bg_p = jnp.pad(bg, (0, 128 - bg.shape[0]), constant_values=f32(-1e30))
    y_out, z_full = _run_tail(y3, idx3, pm, wg_p, _bb(bg_p, 128))
    return z_full[:, :wg.shape[1]], y_out, s_out


# speedup vs baseline: 1.0693x; 1.0693x over previous
import functools

import numpy as np
import jax
import jax.numpy as jnp
from jax.experimental import pallas as pl
from jax.experimental.pallas import tpu as pltpu

_N = 4096
_M = 512
_IN = 128
_HID = 192
_OUT = 128
_K = 6
_NEG = -jnp.inf


def _build_er():
    rng = np.random.default_rng(12345)
    u = rng.random((_M, _M))
    iu = np.triu(np.ones((_M, _M), dtype=bool), 1)
    mask = (u < 0.1) & iu
    r, c = np.nonzero(mask)
    src = np.concatenate([r, c]).astype(np.int32)
    dst = np.concatenate([c, r]).astype(np.int32)
    deg = np.bincount(dst, minlength=_M).astype(np.int32)
    dmax = int(deg.max())
    dpad = -(-dmax // _SB) * _SB
    nbr = np.full((_M, dpad), _M, np.int32)
    fill = np.zeros(_M, np.int32)
    for s, d in zip(src, dst):
        nbr[d, fill[d]] = s
        fill[d] += 1
    steps = dpad // _SB
    nbr_t = nbr.T.reshape(steps, _SB * _M)
    oh = (nbr_t[:, :, None] == np.arange(_M)[None, None, :]).astype(np.float32)
    msk = np.where(nbr_t == _M, -np.inf, 0.0).astype(np.float32)
    msk = np.broadcast_to(msk[:, :, None], (steps, _SB * _M, 128)).copy()
    return oh, msk, steps


_SB = 4
_OH_C, _MSK_C, _STEPS = _build_er()


def _hm_perm(out_dim, nh):
    hs = out_dim // nh
    p = np.empty(out_dim, np.int64)
    for g in range(nh):
        for s in range(hs):
            p[g * hs + s] = s * nh + g
    return p


_P1 = _hm_perm(2 * _HID, 3)
_P2 = _hm_perm(_HID, 2)
_P3 = _hm_perm(_OUT, 2)
_PM = np.zeros((_OUT, _OUT), np.float32)
_PM[np.arange(_OUT), _P3] = 1.0


def _bb(v, w):
    return jnp.broadcast_to(v.reshape(1, w), (8, w))


def _dmon_body(x_ref, w_ref, bc_ref, s_ref, pooled_ref, csum_ref):
    i = pl.program_id(0)
    x = x_ref[...]
    lt = jax.lax.dot_general(w_ref[...], x, (((0,), (1,)), ((), ())),
                             preferred_element_type=jnp.float32)
    lt = lt + bc_ref[:, 0:1]
    m = jnp.max(lt, axis=0, keepdims=True)
    e = jnp.exp(lt - m)
    st = e / jnp.sum(e, axis=0, keepdims=True)
    s_ref[...] = st

    @pl.when(i == 0)
    def _():
        pooled_ref[...] = jnp.zeros_like(pooled_ref)
        csum_ref[...] = jnp.zeros_like(csum_ref)

    pooled_ref[...] += jax.lax.dot_general(
        st, x, (((1,), (0,)), ((), ())), preferred_element_type=jnp.float32)
    csum_ref[...] += jnp.broadcast_to(
        jnp.sum(st, axis=1, keepdims=True), csum_ref.shape)


def _run_dmon(x, w, bc):
    nb = _N // _M
    return pl.pallas_call(
        _dmon_body,
        grid=(nb,),
        in_specs=[
            pl.BlockSpec((_M, _IN), lambda i: (i, 0)),
            pl.BlockSpec((_IN, _M), lambda i: (0, 0)),
            pl.BlockSpec((_M, 128), lambda i: (0, 0)),
        ],
        out_specs=[
            pl.BlockSpec((_M, _M), lambda i: (0, i)),
            pl.BlockSpec((_M, _IN), lambda i: (0, 0)),
            pl.BlockSpec((_M, 128), lambda i: (0, 0)),
        ],
        out_shape=[
            jax.ShapeDtypeStruct((_M, _N), jnp.float32),
            jax.ShapeDtypeStruct((_M, _IN), jnp.float32),
            jax.ShapeDtypeStruct((_M, 128), jnp.float32),
        ],
    )(x, w, bc)


def _onehot(idx1):
    lane = jax.lax.broadcasted_iota(jnp.int32, (_M, _M), 1)
    eq = lane == jnp.broadcast_to(idx1, (_M, _M))
    return jnp.where(eq, jnp.float32(1.0), jnp.float32(0.0))


def _gath(tab, idx1):
    return jnp.dot(_onehot(idx1), tab, preferred_element_type=jnp.float32)


def _conv1_body(pooled_ref, csum_ref, wsrc_ref, wqd_ref, bq_ref,
                wkd_ref, bk_ref, w1d_ref, b1_ref,
                w2p_ref, oh_ref, msk_ref, y1_ref,
                bq_s, bk_s, ud, x0s):
    k = pl.program_id(0)
    d_out = 2 * _HID
    rows = _SB * _M

    @pl.when(k == 0)
    def _():
        pv = pooled_ref[...]
        alpha = jnp.float32(1.6732632423543772)
        scl = jnp.float32(1.0507009873554805)
        selu = scl * jnp.where(pv > 0, pv, alpha * (jnp.exp(pv) - 1.0))
        x0 = selu / csum_ref[...]
        x0s[...] = x0
        bqv = jnp.dot(x0, wqd_ref[...],
                      preferred_element_type=jnp.float32) + bq_ref[0:1, :]
        bkv = jnp.dot(x0, wkd_ref[...],
                      preferred_element_type=jnp.float32) + bk_ref[0:1, :]
        udv = jnp.dot(x0, w1d_ref[...],
                      preferred_element_type=jnp.float32) + b1_ref[0:1, :]
        bq_s[...] = jnp.concatenate([bqv] * _SB, axis=0)
        bk_s[...] = jnp.concatenate([bkv] * _SB, axis=0)
        udx = jnp.concatenate([udv, x0], axis=1)
        ud[...] = jnp.concatenate([udx] * _SB, axis=0)

    oh = oh_ref[0]
    xs = jnp.dot(oh, x0s[...], preferred_element_type=jnp.float32)
    g_all = jnp.dot(xs, wsrc_ref[...], preferred_element_type=jnp.float32)
    q = g_all[:, :d_out] + bq_s[...]
    kk = g_all[:, d_out:2 * d_out] + bk_s[...]
    scale = jnp.sqrt(jnp.float32(d_out))
    hs = 128
    ew = [[jnp.sum(q[:, h * hs:(h + 1) * hs] * kk[:, g * hs:(g + 1) * hs],
                   axis=1, keepdims=True) / scale
           for g in range(3)] for h in range(3)]
    pre = ud[:, 0:_IN] + g_all[:, 2 * d_out:]
    o1 = jax.nn.relu(pre) + ud[:, _IN:2 * _IN]
    m2 = jnp.dot(o1, w2p_ref[...], preferred_element_type=jnp.float32)
    fin = jnp.concatenate(
        [sum(m2[:, h * hs:(h + 1) * hs] * ew[h][g] for h in range(3))
         for g in range(3)], axis=1)
    fin = fin + msk_ref[0, :, 0:1]
    cur = fin[0:_M]
    for s in range(1, _SB):
        cur = jnp.maximum(cur, fin[s * _M:(s + 1) * _M])

    @pl.when(k == 0)
    def _():
        y1_ref[...] = cur

    @pl.when(k > 0)
    def _():
        y1_ref[...] = jnp.maximum(y1_ref[...], cur)


def _run_conv1(pooled, csum, wsrc, wqd, bq, wkd, bk, w1d, b1, w2p, oh_c,
               msk_c):
    d_out = 2 * _HID
    wsrc_w = 2 * d_out + _IN
    rows = _SB * _M
    full = lambda r, c: pl.BlockSpec((r, c), lambda k: (0, 0))
    return pl.pallas_call(
        _conv1_body,
        grid=(_STEPS,),
        in_specs=[
            full(_M, _IN), full(_M, 128),
            full(_IN, wsrc_w), full(_IN, d_out), full(8, d_out),
            full(_IN, d_out), full(8, d_out),
            full(_IN, _IN), full(8, _IN),
            full(_IN, d_out),
            pl.BlockSpec((1, rows, _M), lambda k: (k, 0, 0)),
            pl.BlockSpec((1, rows, 128), lambda k: (k, 0, 0)),
        ],
        out_specs=pl.BlockSpec((_M, d_out), lambda k: (0, 0)),
        out_shape=jax.ShapeDtypeStruct((_M, d_out), jnp.float32),
        scratch_shapes=[
            pltpu.VMEM((rows, d_out), jnp.float32),
            pltpu.VMEM((rows, d_out), jnp.float32),
            pltpu.VMEM((rows, 2 * _IN), jnp.float32),
            pltpu.VMEM((_M, _IN), jnp.float32),
        ],
    )(pooled, csum, wsrc, wqd, bq, wkd, bk, w1d, b1, w2p, oh_c, msk_c)


def _knn_js(y):
    d = y.shape[1]
    sqc = jnp.sum(y * y, axis=1, keepdims=True)
    sqr = jax.lax.dot_general(jnp.ones((1, d), jnp.float32), y * y,
                              (((1,), (1,)), ((), ())),
                              preferred_element_type=jnp.float32)
    g = jax.lax.dot_general(y, y, (((1,), (1,)), ((), ())),
                            preferred_element_type=jnp.float32)
    dist = sqc + sqr - 2.0 * g
    ii = jax.lax.broadcasted_iota(jnp.int32, (_M, _M), 0)
    jj = jax.lax.broadcasted_iota(jnp.int32, (_M, _M), 1)
    dist = jnp.where(ii == jj, jnp.inf, dist)
    js = []
    for _ in range(_K):
        m = jnp.min(dist, axis=1, keepdims=True)
        cand = jnp.where(dist == m, jj, jnp.int32(_N))
        j = jnp.min(cand, axis=1, keepdims=True)
        js.append(j)
        dist = jnp.where(jj == jnp.broadcast_to(j, (_M, _M)), jnp.inf, dist)
    return js


def _convk_body(y_ref, wsrc_ref, wqd_ref, bq_ref, wkd_ref,
                bk_ref, w1d_ref, b1_ref, w2p_ref, nw_ref, nb_ref,
                nms_ref, *refs, din, dout, nh, gather_first, tail):
    y = y_ref[...]
    js = _knn_js(y)
    if not gather_first:
        tsrc = jnp.dot(y, wsrc_ref[...], preferred_element_type=jnp.float32)
    bq = jnp.dot(y, wqd_ref[...],
                 preferred_element_type=jnp.float32) + bq_ref[0:1, :]
    bk = jnp.dot(y, wkd_ref[...],
                 preferred_element_type=jnp.float32) + bk_ref[0:1, :]
    ud = jnp.dot(y, w1d_ref[...],
                 preferred_element_type=jnp.float32) + b1_ref[0:1, :]
    hs = dout // nh
    scale = jnp.sqrt(jnp.float32(dout))
    acc = jnp.zeros((_M, dout), jnp.float32)
    for k in range(_K):
        idx1 = js[k]
        if gather_first:
            ys = _gath(y, idx1)
            g_all = jnp.dot(ys, wsrc_ref[...],
                            preferred_element_type=jnp.float32)
        else:
            g_all = _gath(tsrc, idx1)
        q = g_all[:, :dout] + bq
        kk = g_all[:, dout:2 * dout] + bk
        ew = [[jnp.sum(q[:, h * hs:(h + 1) * hs] * kk[:, g * hs:(g + 1) * hs],
                       axis=1, keepdims=True) / scale
               for g in range(nh)] for h in range(nh)]
        pre = ud + g_all[:, 2 * dout:]
        o1 = jax.nn.relu(pre) + y
        m2 = jnp.dot(o1, w2p_ref[...], preferred_element_type=jnp.float32)
        fin = jnp.concatenate(
            [sum(m2[:, h * hs:(h + 1) * hs] * ew[h][g] for h in range(nh))
             for g in range(nh)], axis=1)
        acc = acc + fin
    ym = acc / jnp.float32(_K)
    mean = jnp.mean(ym, axis=0, keepdims=True)
    out = ym - nms_ref[0:1, :] * mean
    var = jnp.mean(out * out, axis=0, keepdims=True)
    yn = nw_ref[0:1, :] * out / jnp.sqrt(var + 1e-5) + nb_ref[0:1, :]
    if not tail:
        out_ref = refs[0]
        out_ref[...] = yn
        return
    pm_ref, wg_ref, bg_ref, yout_ref, z_ref = refs
    agg = jnp.zeros((_M, _OUT), jnp.float32)
    for k in range(_K):
        agg = agg + _gath(yn, js[k])
    h4 = yn + agg
    yout_ref[...] = jnp.dot(yn, pm_ref[...],
                            preferred_element_type=jnp.float32)
    logits = jnp.dot(h4, wg_ref[...],
                     preferred_element_type=jnp.float32) + bg_ref[0:1, :]
    m = jnp.max(logits, axis=1, keepdims=True)
    e = jnp.exp(logits - m)
    z_ref[...] = e / jnp.sum(e, axis=1, keepdims=True)


def _run_convk(y, wsrc, wqd, bq, wkd, bk, w1d, b1, w2p,
               nw, nb, nms, din, dout, nh, gather_first):
    body = functools.partial(_convk_body, din=din, dout=dout, nh=nh,
                             gather_first=gather_first, tail=False)
    return pl.pallas_call(
        body,
        out_shape=jax.ShapeDtypeStruct((_M, dout), jnp.float32),
    )(y, wsrc, wqd, bq, wkd, bk, w1d, b1, w2p, nw, nb, nms)


def _run_convk_tail(y, wsrc, wqd, bq, wkd, bk, w1d, b1, w2p,
                    nw, nb, nms, pm, wg, bg, din, dout, nh):
    body = functools.partial(_convk_body, din=din, dout=dout, nh=nh,
                             gather_first=True, tail=True)
    return pl.pallas_call(
        body,
        out_shape=[
            jax.ShapeDtypeStruct((_M, _OUT), jnp.float32),
            jax.ShapeDtypeStruct((_M, 128), jnp.float32),
        ],
    )(y, wsrc, wqd, bq, wkd, bk, w1d, b1, w2p, nw, nb, nms, pm, wg, bg)


def kernel(x, edge_index, batch, params):
    p = params
    del edge_index, batch
    f32 = jnp.float32
    p1, p2, p3 = _P1, _P2, _P3

    w_dmon = p['dmon_w1'] @ p['dmon_w2']
    b_dmon = p['dmon_b1'] @ p['dmon_w2'] + p['dmon_b2']
    bc = jnp.broadcast_to(b_dmon[:, None], (_M, 128))

    s_out, pooled, csum = _run_dmon(x, w_dmon, bc)

    d1 = 2 * _HID
    wq = p['c1_wq1'] @ p['c1_wq2']
    bq = p['c1_bq1'] @ p['c1_wq2'] + p['c1_bq2']
    wk = p['c1_wk1'] @ p['c1_wk2']
    bk = p['c1_bk1'] @ p['c1_wk2'] + p['c1_bk2']
    w2p1 = p['c1_w2'][:, p1]
    wsrc1 = jnp.concatenate([wq[:_IN], wk[:_IN], p['c1_w1'][_IN:]], axis=1)
    y1 = _run_conv1(
        pooled, csum,
        wsrc1, wq[_IN:], _bb(bq, d1),
        wk[_IN:], _bb(bk, d1),
        p['c1_w1'][:_IN], _bb(p['c1_b1'], _IN),
        w2p1, jnp.asarray(_OH_C), jnp.asarray(_MSK_C))

    wsrc2 = jnp.concatenate(
        [p['c2_wq'][:d1][p1], p['c2_wk'][:d1][p1],
         p['c2_w1'][d1:][p1][:, p1]], axis=1)
    y2 = _run_convk(
        y1,
        wsrc2, p['c2_wq'][d1:][p1], _bb(p['c2_bq'], _HID),
        p['c2_wk'][d1:][p1], _bb(p['c2_bk'], _HID),
        p['c2_w1'][:d1][p1][:, p1],
        _bb(p['c2_b1'][p1], d1),
        p['c2_w2'][p1][:, p2],
        _bb(p['n2_w'][p2], _HID), _bb(p['n2_b'][p2], _HID),
        _bb(p['n2_ms'][p2], _HID),
        din=d1, dout=_HID, nh=2, gather_first=False)

    wg = p['g_w1'] @ p['g_w2']
    bg = p['g_b1'] @ p['g_w2'] + p['g_b2']
    pm = jnp.asarray(_PM)
    wg_p = jnp.pad(pm @ wg, ((0, 0), (0, 128 - wg.shape[1])))
    bg_p = jnp.pad(bg, (0, 128 - bg.shape[0]), constant_values=f32(-1e30))
    wsrc3 = jnp.concatenate(
        [p['c3_wq'][:_HID][p2], p['c3_wk'][:_HID][p2],
         p['c3_w1'][_HID:][p2][:, p2]], axis=1)
    y_out, z_full = _run_convk_tail(
        y2,
        wsrc3, p['c3_wq'][_HID:][p2], _bb(p['c3_bq'], _OUT),
        p['c3_wk'][_HID:][p2], _bb(p['c3_bk'], _OUT),
        p['c3_w1'][:_HID][p2][:, p2],
        _bb(p['c3_b1'][p2], _HID),
        p['c3_w2'][p2][:, p3],
        _bb(p['n3_w'][p3], _OUT), _bb(p['n3_b'][p3], _OUT),
        _bb(p['n3_ms'][p3], _OUT),
        pm, wg_p, _bb(bg_p, 128),
        din=_HID, dout=_OUT, nh=2)
    return z_full[:, :wg.shape[1]], y_out, s_out
